# baseline (speedup 1.0000x reference)
import dataclasses
import functools

import jax
import jax.numpy as jnp
from jax import lax
from jax.experimental import pallas as pl
from jax.experimental.pallas import tpu as pltpu
from jax.experimental.pallas import tpu_sc as plsc

N_MAX_OBJECTS = 1000
THRESHOLD_SCORE = 0.5
AREA_LIMIT = 1024
NPAD = 20480
NTILES = 16
EPT = NPAD // NTILES
VPT = EPT // 16
TABLE_WORDS = 96


def _area_kernel(mt_ref, out_ref, acc_ref, *, n_steps):
    step = pl.program_id(0)

    @pl.when(step == 0)
    def _init():
        acc_ref[...] = jnp.zeros_like(acc_ref)

    m = mt_ref[...]
    nz = (m != 0).astype(jnp.int32).reshape(-1, 8, m.shape[1])
    acc_ref[...] += jnp.sum(nz, axis=0)

    @pl.when(step == n_steps - 1)
    def _fin():
        total = jnp.sum(acc_ref[...], axis=0, keepdims=True)
        out_ref[...] = (total > AREA_LIMIT).astype(jnp.int32)


def _front_sc_kernel(scores_hbm, labels_hbm, classes_hbm, out_hbm,
                     sv, lv, ov, table, clsv, mine, mbuf, shared,
                     *, n_classes):
    sid = lax.axis_index("s")
    base = sid * EPT
    pltpu.sync_copy(scores_hbm.at[pl.ds(base, EPT)], sv)
    pltpu.sync_copy(labels_hbm.at[pl.ds(base, EPT)], lv)
    pltpu.sync_copy(classes_hbm, clsv)

    lane = lax.iota(jnp.int32, 16)
    zero16 = jnp.zeros((16,), jnp.int32)
    one16 = jnp.ones((16,), jnp.int32)

    for j in range(TABLE_WORDS // 16):
        table[pl.ds(j * 16, 16)] = zero16
    plsc.store_scatter(table, [clsv[pl.ds(0, 16)]], one16,
                       mask=lane < min(16, n_classes))
    plsc.store_scatter(table, [clsv[pl.ds(16, 16)]], one16,
                       mask=lane < (n_classes - 16))

    def _global_sum(acc, parity):
        mine[...] = acc
        off = parity * (NTILES * 16)
        pltpu.sync_copy(mine, shared.at[pl.ds(off + sid * 16, 16)])
        plsc.subcore_barrier()
        pltpu.sync_copy(shared.at[pl.ds(off, NTILES * 16)], mbuf)
        tot = zero16
        for t in range(NTILES):
            tot = tot + mbuf[pl.ds(t * 16, 16)]
        return jnp.sum(tot)

    def _count_ge(mid):
        mid_v = jnp.full((16,), mid, jnp.int32)
        acc = zero16
        for j in range(VPT):
            kj = plsc.bitcast(sv[pl.ds(j * 16, 16)], jnp.int32)
            acc = acc + (kj >= mid_v).astype(jnp.int32)
        return acc

    def body(r, carry):
        lo, hi = carry
        mid = (lo + hi) // 2
        take = _global_sum(_count_ge(mid), r % 2) >= N_MAX_OBJECTS
        return (jnp.where(take, mid, lo), jnp.where(take, hi, mid))

    lo, _ = lax.fori_loop(0, 31, body, (jnp.int32(0), jnp.int32(1 << 30)))
    thresh = lo
    thresh_v = jnp.full((16,), thresh, jnp.int32)

    acc_gt = zero16
    acc_eq = zero16
    for j in range(VPT):
        kj = plsc.bitcast(sv[pl.ds(j * 16, 16)], jnp.int32)
        acc_gt = acc_gt + (kj > thresh_v).astype(jnp.int32)
        acc_eq = acc_eq + (kj == thresh_v).astype(jnp.int32)
    cnt_gt = _global_sum(acc_gt, 1)
    k_rem = jnp.int32(N_MAX_OBJECTS) - cnt_gt
    k_rem_v = jnp.full((16,), k_rem, jnp.int32)

    my_eq = jnp.sum(acc_eq)
    mine[...] = jnp.full((16,), my_eq, jnp.int32)
    pltpu.sync_copy(mine, shared.at[pl.ds(sid * 16, 16)])
    plsc.subcore_barrier()
    pltpu.sync_copy(shared.at[pl.ds(0, NTILES * 16)], mbuf)
    run = zero16
    for t in range(NTILES):
        take_t = (jnp.int32(t) < sid).astype(jnp.int32)
        run = run + mbuf[pl.ds(t * 16, 16)] * jnp.full((16,), take_t, jnp.int32)

    half = jnp.full((16,), THRESHOLD_SCORE, jnp.float32)
    for j in range(VPT):
        sj = sv[pl.ds(j * 16, 16)]
        kj = plsc.bitcast(sj, jnp.int32)
        eq = (kj == thresh_v).astype(jnp.int32)
        excl = run + plsc.cumsum(eq) - eq
        topk = (kj > thresh_v) | ((eq > 0) & (excl < k_rem_v))
        run = run + jnp.full((16,), jnp.sum(eq), jnp.int32)
        member = plsc.load_gather(table, [lv[pl.ds(j * 16, 16)]])
        front = topk & (sj > half) & (member > 0)
        ov[pl.ds(j * 16, 16)] = front.astype(jnp.int32)

    pltpu.sync_copy(ov, out_hbm.at[pl.ds(base, EPT)])


def _front_sc(scores_pad, labels_pad, classes_pad, n_classes):
    mesh = plsc.VectorSubcoreMesh(
        core_axis_name="c", subcore_axis_name="s", num_cores=1)
    cp = pltpu.CompilerParams()
    if "needs_layout_passes" in pltpu.CompilerParams.__dataclass_fields__:
        cp = dataclasses.replace(cp, needs_layout_passes=False)
    kern = functools.partial(
        pl.kernel,
        out_type=jax.ShapeDtypeStruct((NPAD,), jnp.int32),
        mesh=mesh,
        compiler_params=cp,
        scratch_types=[
            pltpu.VMEM((EPT,), jnp.float32),
            pltpu.VMEM((EPT,), jnp.int32),
            pltpu.VMEM((EPT,), jnp.int32),
            pltpu.VMEM((TABLE_WORDS,), jnp.int32),
            pltpu.VMEM((32,), jnp.int32),
            pltpu.VMEM((16,), jnp.int32),
            pltpu.VMEM((NTILES * 16,), jnp.int32),
            pltpu.VMEM_SHARED((2 * NTILES * 16,), jnp.int32),
        ],
    )(functools.partial(_front_sc_kernel, n_classes=n_classes))
    return kern(scores_pad, labels_pad, classes_pad)


def kernel(labels, scores, masks, all_classes):
    n = scores.shape[0]
    n_classes = all_classes.shape[0]
    _, h, w = masks.shape
    hw = h * w

    scores_pad = jnp.concatenate(
        [scores, jnp.zeros((NPAD - n,), jnp.float32)])
    labels_pad = jnp.concatenate(
        [labels, jnp.zeros((NPAD - n,), jnp.int32)])
    classes_pad = jnp.zeros((32,), jnp.int32).at[:n_classes].set(all_classes)

    masks_t = jnp.transpose(masks, (1, 2, 0)).reshape(hw, n)
    br = 256
    n_steps = hw // br
    area2d = pl.pallas_call(
        functools.partial(_area_kernel, n_steps=n_steps),
        grid=(n_steps,),
        in_specs=[pl.BlockSpec((br, n), lambda i: (i, 0))],
        out_specs=pl.BlockSpec((1, n), lambda i: (0, 0)),
        out_shape=jax.ShapeDtypeStruct((1, n), jnp.int32),
        scratch_shapes=[pltpu.VMEM((8, n), jnp.int32)],
    )(masks_t)
    front = _front_sc(scores_pad, labels_pad, classes_pad, n_classes)
    return (front[:n] > 0) & (area2d.reshape(n) > 0)


# speedup vs baseline: 1.0012x; 1.0012x over previous
import dataclasses
import functools

import jax
import jax.numpy as jnp
from jax import lax
from jax.experimental import pallas as pl
from jax.experimental.pallas import tpu as pltpu
from jax.experimental.pallas import tpu_sc as plsc

N_MAX_OBJECTS = 1000
THRESHOLD_SCORE = 0.5
AREA_LIMIT = 1024
NPAD = 20480
NTILES = 16
EPT = NPAD // NTILES
VPT = EPT // 16
TABLE_WORDS = 96


def _area_kernel(mt_ref, out_ref, acc_ref, *, n_steps):
    step = pl.program_id(0)

    @pl.when(step == 0)
    def _init():
        acc_ref[...] = jnp.zeros_like(acc_ref)

    m = mt_ref[...]
    nz = (m != 0).astype(jnp.int32).reshape(-1, 8, m.shape[1])
    acc_ref[...] += jnp.sum(nz, axis=0)

    @pl.when(step == n_steps - 1)
    def _fin():
        total = jnp.sum(acc_ref[...], axis=0, keepdims=True)
        out_ref[...] = (total > AREA_LIMIT).astype(jnp.int32)


def _front_sc_kernel(scores_hbm, labels_hbm, classes_hbm, out_hbm,
                     sv, lv, ov, table, clsv, mine, mbuf, shared, cnt,
                     *, n_classes):
    sid = lax.axis_index("s")
    base = sid * EPT
    pltpu.sync_copy(scores_hbm.at[pl.ds(base, EPT)], sv)
    pltpu.sync_copy(labels_hbm.at[pl.ds(base, EPT)], lv)
    pltpu.sync_copy(classes_hbm, clsv)

    lane = lax.iota(jnp.int32, 16)
    zero16 = jnp.zeros((16,), jnp.int32)
    one16 = jnp.ones((16,), jnp.int32)

    for j in range(TABLE_WORDS // 16):
        table[pl.ds(j * 16, 16)] = zero16
    plsc.store_scatter(table, [clsv[pl.ds(0, 16)]], one16,
                       mask=lane < min(16, n_classes))
    plsc.store_scatter(table, [clsv[pl.ds(16, 16)]], one16,
                       mask=lane < (n_classes - 16))

    def _global_sum(acc, slot):
        plsc.fetch_and_add(cnt.at[slot], jnp.sum(acc), subcore_id=0)
        plsc.subcore_barrier()
        return plsc.fetch_and_add(cnt.at[slot], 0, subcore_id=0)

    def _count_ge(mid):
        mid_v = jnp.full((16,), mid, jnp.int32)
        acc = zero16
        for j in range(VPT):
            kj = plsc.bitcast(sv[pl.ds(j * 16, 16)], jnp.int32)
            acc = acc + (kj >= mid_v).astype(jnp.int32)
        return acc

    def body(r, carry):
        lo, hi = carry
        mid = (lo + hi) // 2
        take = _global_sum(_count_ge(mid), r) >= N_MAX_OBJECTS
        return (jnp.where(take, mid, lo), jnp.where(take, hi, mid))

    lo, _ = lax.fori_loop(0, 31, body, (jnp.int32(0), jnp.int32(1 << 30)))
    thresh = lo
    thresh_v = jnp.full((16,), thresh, jnp.int32)

    acc_gt = zero16
    acc_eq = zero16
    for j in range(VPT):
        kj = plsc.bitcast(sv[pl.ds(j * 16, 16)], jnp.int32)
        acc_gt = acc_gt + (kj > thresh_v).astype(jnp.int32)
        acc_eq = acc_eq + (kj == thresh_v).astype(jnp.int32)
    cnt_gt = _global_sum(acc_gt, 31)
    k_rem = jnp.int32(N_MAX_OBJECTS) - cnt_gt
    k_rem_v = jnp.full((16,), k_rem, jnp.int32)

    my_eq = jnp.sum(acc_eq)
    mine[...] = jnp.full((16,), my_eq, jnp.int32)
    pltpu.sync_copy(mine, shared.at[pl.ds(sid * 16, 16)])
    plsc.subcore_barrier()
    pltpu.sync_copy(shared.at[pl.ds(0, NTILES * 16)], mbuf)
    run = zero16
    for t in range(NTILES):
        take_t = (jnp.int32(t) < sid).astype(jnp.int32)
        run = run + mbuf[pl.ds(t * 16, 16)] * jnp.full((16,), take_t, jnp.int32)

    half = jnp.full((16,), THRESHOLD_SCORE, jnp.float32)
    for j in range(VPT):
        sj = sv[pl.ds(j * 16, 16)]
        kj = plsc.bitcast(sj, jnp.int32)
        eq = (kj == thresh_v).astype(jnp.int32)
        excl = run + plsc.cumsum(eq) - eq
        topk = (kj > thresh_v) | ((eq > 0) & (excl < k_rem_v))
        run = run + jnp.full((16,), jnp.sum(eq), jnp.int32)
        member = plsc.load_gather(table, [lv[pl.ds(j * 16, 16)]])
        front = topk & (sj > half) & (member > 0)
        ov[pl.ds(j * 16, 16)] = front.astype(jnp.int32)

    pltpu.sync_copy(ov, out_hbm.at[pl.ds(base, EPT)])


def _front_sc(scores_pad, labels_pad, classes_pad, n_classes):
    mesh = plsc.VectorSubcoreMesh(
        core_axis_name="c", subcore_axis_name="s", num_cores=1)
    cp = pltpu.CompilerParams()
    if "needs_layout_passes" in pltpu.CompilerParams.__dataclass_fields__:
        cp = dataclasses.replace(cp, needs_layout_passes=False)
    kern = functools.partial(
        pl.kernel,
        out_type=jax.ShapeDtypeStruct((NPAD,), jnp.int32),
        mesh=mesh,
        compiler_params=cp,
        scratch_types=[
            pltpu.VMEM((EPT,), jnp.float32),
            pltpu.VMEM((EPT,), jnp.int32),
            pltpu.VMEM((EPT,), jnp.int32),
            pltpu.VMEM((TABLE_WORDS,), jnp.int32),
            pltpu.VMEM((32,), jnp.int32),
            pltpu.VMEM((16,), jnp.int32),
            pltpu.VMEM((NTILES * 16,), jnp.int32),
            pltpu.VMEM_SHARED((2 * NTILES * 16,), jnp.int32),
            pltpu.SMEM((40,), jnp.int32),
        ],
    )(functools.partial(_front_sc_kernel, n_classes=n_classes))
    return kern(scores_pad, labels_pad, classes_pad)


def kernel(labels, scores, masks, all_classes):
    n = scores.shape[0]
    n_classes = all_classes.shape[0]
    _, h, w = masks.shape
    hw = h * w

    scores_pad = jnp.concatenate(
        [scores, jnp.zeros((NPAD - n,), jnp.float32)])
    labels_pad = jnp.concatenate(
        [labels, jnp.zeros((NPAD - n,), jnp.int32)])
    classes_pad = jnp.zeros((32,), jnp.int32).at[:n_classes].set(all_classes)

    masks_t = jnp.transpose(masks, (1, 2, 0)).reshape(hw, n)
    br = 256
    n_steps = hw // br
    area2d = pl.pallas_call(
        functools.partial(_area_kernel, n_steps=n_steps),
        grid=(n_steps,),
        in_specs=[pl.BlockSpec((br, n), lambda i: (i, 0))],
        out_specs=pl.BlockSpec((1, n), lambda i: (0, 0)),
        out_shape=jax.ShapeDtypeStruct((1, n), jnp.int32),
        scratch_shapes=[pltpu.VMEM((8, n), jnp.int32)],
    )(masks_t)
    front = _front_sc(scores_pad, labels_pad, classes_pad, n_classes)
    return (front[:n] > 0) & (area2d.reshape(n) > 0)


# speedup vs baseline: 1.0092x; 1.0081x over previous
import dataclasses
import functools

import jax
import jax.numpy as jnp
from jax import lax
from jax.experimental import pallas as pl
from jax.experimental.pallas import tpu as pltpu
from jax.experimental.pallas import tpu_sc as plsc

N_MAX_OBJECTS = 1000
THRESHOLD_SCORE = 0.5
AREA_LIMIT = 1024
NPAD = 20480
NTILES = 16
EPT = NPAD // NTILES
VPT = EPT // 16
TABLE_WORDS = 96
HALF_KEY = 0x3F000000


def _area_kernel(mt_ref, out_ref, acc_ref, *, n_steps):
    step = pl.program_id(0)

    @pl.when(step == 0)
    def _init():
        acc_ref[...] = jnp.zeros_like(acc_ref)

    m = mt_ref[...]
    nz = (m != 0).astype(jnp.int32).reshape(-1, 8, m.shape[1])
    acc_ref[...] += jnp.sum(nz, axis=0)

    @pl.when(step == n_steps - 1)
    def _fin():
        total = jnp.sum(acc_ref[...], axis=0, keepdims=True)
        out_ref[...] = (total > AREA_LIMIT).astype(jnp.int32)


def _front_sc_kernel(packed_hbm, out_hbm,
                     sv, lv, ov, table, clsv, mine, mbuf, shared, cnt,
                     *, n_classes):
    sid = lax.axis_index("s")
    base = sid * EPT
    pltpu.sync_copy(packed_hbm.at[pl.ds(base, EPT)], sv)
    pltpu.sync_copy(packed_hbm.at[pl.ds(NPAD + base, EPT)], lv)
    pltpu.sync_copy(packed_hbm.at[pl.ds(2 * NPAD, 32)], clsv)

    lane = lax.iota(jnp.int32, 16)
    zero16 = jnp.zeros((16,), jnp.int32)
    one16 = jnp.ones((16,), jnp.int32)

    for j in range(TABLE_WORDS // 16):
        table[pl.ds(j * 16, 16)] = zero16
    plsc.store_scatter(table, [clsv[pl.ds(0, 16)]], one16,
                       mask=lane < min(16, n_classes))
    plsc.store_scatter(table, [clsv[pl.ds(16, 16)]], one16,
                       mask=lane < (n_classes - 16))

    def _global_sum(acc, slot):
        plsc.fetch_and_add(cnt.at[slot], jnp.sum(acc), subcore_id=0)
        plsc.subcore_barrier()
        return plsc.fetch_and_add(cnt.at[slot], 0, subcore_id=0)

    def _count_ge(mid):
        mid_v = jnp.full((16,), mid, jnp.int32)
        acc = zero16
        for j in range(VPT):
            acc = acc + (sv[pl.ds(j * 16, 16)] >= mid_v).astype(jnp.int32)
        return acc

    def body(r, carry):
        lo, hi = carry
        mid = (lo + hi) // 2
        take = _global_sum(_count_ge(mid), r) >= N_MAX_OBJECTS
        return (jnp.where(take, mid, lo), jnp.where(take, hi, mid))

    lo, _ = lax.fori_loop(0, 31, body, (jnp.int32(0), jnp.int32(1 << 30)))
    thresh = lo
    thresh_v = jnp.full((16,), thresh, jnp.int32)

    acc_gt = zero16
    acc_eq = zero16
    for j in range(VPT):
        kj = sv[pl.ds(j * 16, 16)]
        acc_gt = acc_gt + (kj > thresh_v).astype(jnp.int32)
        acc_eq = acc_eq + (kj == thresh_v).astype(jnp.int32)
    cnt_gt = _global_sum(acc_gt, 31)
    k_rem = jnp.int32(N_MAX_OBJECTS) - cnt_gt
    k_rem_v = jnp.full((16,), k_rem, jnp.int32)

    my_eq = jnp.sum(acc_eq)
    mine[...] = jnp.full((16,), my_eq, jnp.int32)
    pltpu.sync_copy(mine, shared.at[pl.ds(sid * 16, 16)])
    plsc.subcore_barrier()
    pltpu.sync_copy(shared.at[pl.ds(0, NTILES * 16)], mbuf)
    run = zero16
    for t in range(NTILES):
        take_t = (jnp.int32(t) < sid).astype(jnp.int32)
        run = run + mbuf[pl.ds(t * 16, 16)] * jnp.full((16,), take_t, jnp.int32)

    half_key = jnp.full((16,), HALF_KEY, jnp.int32)
    for j in range(VPT):
        kj = sv[pl.ds(j * 16, 16)]
        eq = (kj == thresh_v).astype(jnp.int32)
        excl = run + plsc.cumsum(eq) - eq
        topk = (kj > thresh_v) | ((eq > 0) & (excl < k_rem_v))
        run = run + jnp.full((16,), jnp.sum(eq), jnp.int32)
        member = plsc.load_gather(table, [lv[pl.ds(j * 16, 16)]])
        front = topk & (kj > half_key) & (member > 0)
        ov[pl.ds(j * 16, 16)] = front.astype(jnp.int32)

    pltpu.sync_copy(ov, out_hbm.at[pl.ds(base, EPT)])


def _front_sc(packed, n_classes):
    mesh = plsc.VectorSubcoreMesh(
        core_axis_name="c", subcore_axis_name="s", num_cores=1)
    cp = pltpu.CompilerParams()
    if "needs_layout_passes" in pltpu.CompilerParams.__dataclass_fields__:
        cp = dataclasses.replace(cp, needs_layout_passes=False)
    kern = functools.partial(
        pl.kernel,
        out_type=jax.ShapeDtypeStruct((NPAD,), jnp.int32),
        mesh=mesh,
        compiler_params=cp,
        scratch_types=[
            pltpu.VMEM((EPT,), jnp.int32),
            pltpu.VMEM((EPT,), jnp.int32),
            pltpu.VMEM((EPT,), jnp.int32),
            pltpu.VMEM((TABLE_WORDS,), jnp.int32),
            pltpu.VMEM((32,), jnp.int32),
            pltpu.VMEM((16,), jnp.int32),
            pltpu.VMEM((NTILES * 16,), jnp.int32),
            pltpu.VMEM_SHARED((2 * NTILES * 16,), jnp.int32),
            pltpu.SMEM((40,), jnp.int32),
        ],
    )(functools.partial(_front_sc_kernel, n_classes=n_classes))
    return kern(packed)


def kernel(labels, scores, masks, all_classes):
    n = scores.shape[0]
    n_classes = all_classes.shape[0]
    _, h, w = masks.shape
    hw = h * w

    keys = lax.bitcast_convert_type(scores, jnp.int32)
    zpad = jnp.zeros((NPAD - n,), jnp.int32)
    packed = jnp.concatenate([
        keys, zpad, labels, zpad,
        jnp.zeros((32,), jnp.int32).at[:n_classes].set(all_classes)])

    masks_t = jnp.transpose(masks, (1, 2, 0)).reshape(hw, n)
    br = 256
    n_steps = hw // br
    area2d = pl.pallas_call(
        functools.partial(_area_kernel, n_steps=n_steps),
        grid=(n_steps,),
        in_specs=[pl.BlockSpec((br, n), lambda i: (i, 0))],
        out_specs=pl.BlockSpec((1, n), lambda i: (0, 0)),
        out_shape=jax.ShapeDtypeStruct((1, n), jnp.int32),
        scratch_shapes=[pltpu.VMEM((8, n), jnp.int32)],
    )(masks_t)
    front = _front_sc(packed, n_classes)
    return (front[:n] > 0) & (area2d.reshape(n) > 0)


# speedup vs baseline: 1.0182x; 1.0089x over previous
import dataclasses
import functools

import jax
import jax.numpy as jnp
from jax import lax
from jax.experimental import pallas as pl
from jax.experimental.pallas import tpu as pltpu
from jax.experimental.pallas import tpu_sc as plsc

N_MAX_OBJECTS = 1000
THRESHOLD_SCORE = 0.5
AREA_LIMIT = 1024
NPAD = 20480
NTILES = 16
EPT = NPAD // NTILES
VPT = EPT // 16
TABLE_WORDS = 96
HALF_KEY = 0x3F000000


def _area_kernel(mt_ref, out_ref, acc_ref, *, n_steps):
    step = pl.program_id(0)

    @pl.when(step == 0)
    def _init():
        acc_ref[...] = jnp.zeros_like(acc_ref)

    m = mt_ref[...]
    nz = (m != 0).astype(jnp.int32).reshape(-1, 8, m.shape[1])
    acc_ref[...] += jnp.sum(nz, axis=0)

    @pl.when(step == n_steps - 1)
    def _fin():
        total = jnp.sum(acc_ref[...], axis=0, keepdims=True)
        out_ref[...] = (total > AREA_LIMIT).astype(jnp.int32)


def _front_sc_kernel(packed_hbm, out_hbm,
                     sv, lv, ov, table, clsv, mine, mbuf, shared, cnt,
                     *, n_classes):
    sid = lax.axis_index("s")
    base = sid * EPT
    pltpu.sync_copy(packed_hbm.at[pl.ds(base, EPT)], sv)
    pltpu.sync_copy(packed_hbm.at[pl.ds(NPAD + base, EPT)], lv)
    pltpu.sync_copy(packed_hbm.at[pl.ds(2 * NPAD, 32)], clsv)

    lane = lax.iota(jnp.int32, 16)
    zero16 = jnp.zeros((16,), jnp.int32)
    one16 = jnp.ones((16,), jnp.int32)

    for j in range(TABLE_WORDS // 16):
        table[pl.ds(j * 16, 16)] = zero16
    plsc.store_scatter(table, [clsv[pl.ds(0, 16)]], one16,
                       mask=lane < min(16, n_classes))
    plsc.store_scatter(table, [clsv[pl.ds(16, 16)]], one16,
                       mask=lane < (n_classes - 16))

    def _global_sum(acc, slot):
        plsc.fetch_and_add(cnt.at[slot], jnp.sum(acc), subcore_id=0)
        plsc.subcore_barrier()
        return plsc.fetch_and_add(cnt.at[slot], 0, subcore_id=0)

    def _count_ge(mid):
        mid_v = jnp.full((16,), mid, jnp.int32)
        acc = zero16
        for j in range(VPT):
            acc = acc + (sv[pl.ds(j * 16, 16)] >= mid_v).astype(jnp.int32)
        return acc

    def body(r, carry):
        lo, hi = carry
        mid = (lo + hi) // 2
        take = _global_sum(_count_ge(mid), r) >= N_MAX_OBJECTS
        return (jnp.where(take, mid, lo), jnp.where(take, hi, mid))

    lo, _ = lax.fori_loop(0, 31, body, (jnp.int32(0), jnp.int32(1 << 30)))
    thresh = lo
    thresh_v = jnp.full((16,), thresh, jnp.int32)

    acc_gt = zero16
    acc_eq = zero16
    for j in range(VPT):
        kj = sv[pl.ds(j * 16, 16)]
        acc_gt = acc_gt + (kj > thresh_v).astype(jnp.int32)
        acc_eq = acc_eq + (kj == thresh_v).astype(jnp.int32)
    cnt_gt = _global_sum(acc_gt, 31)
    k_rem = jnp.int32(N_MAX_OBJECTS) - cnt_gt
    k_rem_v = jnp.full((16,), k_rem, jnp.int32)

    my_eq = jnp.sum(acc_eq)
    mine[...] = jnp.full((16,), my_eq, jnp.int32)
    pltpu.sync_copy(mine, shared.at[pl.ds(sid * 16, 16)])
    plsc.subcore_barrier()
    pltpu.sync_copy(shared.at[pl.ds(0, NTILES * 16)], mbuf)
    run = zero16
    for t in range(NTILES):
        take_t = (jnp.int32(t) < sid).astype(jnp.int32)
        run = run + mbuf[pl.ds(t * 16, 16)] * jnp.full((16,), take_t, jnp.int32)

    half_key = jnp.full((16,), HALF_KEY, jnp.int32)
    for j in range(VPT):
        kj = sv[pl.ds(j * 16, 16)]
        eq = (kj == thresh_v).astype(jnp.int32)
        excl = run + plsc.cumsum(eq) - eq
        topk = (kj > thresh_v) | ((eq > 0) & (excl < k_rem_v))
        run = run + jnp.full((16,), jnp.sum(eq), jnp.int32)
        member = plsc.load_gather(table, [lv[pl.ds(j * 16, 16)]])
        front = topk & (kj > half_key) & (member > 0)
        ov[pl.ds(j * 16, 16)] = front.astype(jnp.int32)

    pltpu.sync_copy(ov, out_hbm.at[pl.ds(base, EPT)])


def _front_sc(packed, n_classes):
    mesh = plsc.VectorSubcoreMesh(
        core_axis_name="c", subcore_axis_name="s", num_cores=1)
    cp = pltpu.CompilerParams()
    if "needs_layout_passes" in pltpu.CompilerParams.__dataclass_fields__:
        cp = dataclasses.replace(cp, needs_layout_passes=False)
    kern = functools.partial(
        pl.kernel,
        out_type=jax.ShapeDtypeStruct((NPAD,), jnp.int32),
        mesh=mesh,
        compiler_params=cp,
        scratch_types=[
            pltpu.VMEM((EPT,), jnp.int32),
            pltpu.VMEM((EPT,), jnp.int32),
            pltpu.VMEM((EPT,), jnp.int32),
            pltpu.VMEM((TABLE_WORDS,), jnp.int32),
            pltpu.VMEM((32,), jnp.int32),
            pltpu.VMEM((16,), jnp.int32),
            pltpu.VMEM((NTILES * 16,), jnp.int32),
            pltpu.VMEM_SHARED((2 * NTILES * 16,), jnp.int32),
            pltpu.SMEM((40,), jnp.int32),
        ],
    )(functools.partial(_front_sc_kernel, n_classes=n_classes))
    return kern(packed)


def kernel(labels, scores, masks, all_classes):
    n = scores.shape[0]
    n_classes = all_classes.shape[0]
    _, h, w = masks.shape
    hw = h * w

    keys = lax.bitcast_convert_type(scores, jnp.int32)
    zpad = jnp.zeros((NPAD - n,), jnp.int32)
    packed = jnp.concatenate([
        keys, zpad, labels, zpad,
        jnp.zeros((32,), jnp.int32).at[:n_classes].set(all_classes)])

    masks_t = jnp.transpose(masks, (1, 2, 0)).reshape(hw, n)
    br = 128
    n_steps = hw // br
    area2d = pl.pallas_call(
        functools.partial(_area_kernel, n_steps=n_steps),
        grid=(n_steps,),
        in_specs=[pl.BlockSpec((br, n), lambda i: (i, 0))],
        out_specs=pl.BlockSpec((1, n), lambda i: (0, 0)),
        out_shape=jax.ShapeDtypeStruct((1, n), jnp.int32),
        scratch_shapes=[pltpu.VMEM((8, n), jnp.int32)],
    )(masks_t)
    front = _front_sc(packed, n_classes)
    return (front[:n] > 0) & (area2d.reshape(n) > 0)
